# baseline (speedup 1.0000x reference)
import functools

import jax
import jax.numpy as jnp
from jax import lax
from jax.experimental import pallas as pl
from jax.experimental.pallas import tpu as pltpu
from jax.experimental.pallas import tpu_sc as plsc

_N = 10000
_E = 160000
_D = 256
_H = 256
_H2 = _H // 2

_NC = 2
_NS = 16
_NW = _NC * _NS
_K = 80
_EPT = _E // _NS
_NCHUNK = _EPT // _K
_RPT = _N // _NS
_EPW = _E // _NW

_B = 1024
_G = (_N + _B - 1) // _B


def _deg_body(col_hbm, parts_hbm, colbuf, acc):
    cid = lax.axis_index("c")
    sid = lax.axis_index("s")
    wid = cid * _NS + sid

    zeros16 = jnp.zeros((16,), jnp.float32)

    def zstep(j, carry):
        acc[pl.ds(j * 16, 16)] = zeros16
        return carry

    lax.fori_loop(0, _N // 16, zstep, 0)

    colbuf[pl.ds(_EPW - 8, 16)] = jnp.zeros((16,), jnp.int32)
    pltpu.sync_copy(col_hbm.at[pl.ds(wid * _EPW, _EPW)], colbuf.at[pl.ds(0, _EPW)])

    ones16 = jnp.ones((16,), jnp.float32)
    nfull = _EPW // 16

    def estep(i, carry):
        idx = colbuf[pl.ds(i * 16, 16)]
        plsc.addupdate_scatter(acc, [idx], ones16)
        return carry

    lax.fori_loop(0, nfull, estep, 0)

    rem = _EPW - nfull * 16
    idx = colbuf[pl.ds(nfull * 16, 16)]
    mask = lax.iota(jnp.int32, 16) < rem
    plsc.addupdate_scatter(acc, [idx], ones16, mask=mask)

    pltpu.sync_copy(acc, parts_hbm.at[wid])


_NB = 3
_LOOK = 1


def _agg_body(ztA, ztB, rowr, colr, aggA, aggB, idxrow, idxcol, gbuf, acc,
              sems):
    cid = lax.axis_index("c")
    sid = lax.axis_index("s")

    def run(zt, agg):
        def g_issue(b, j):
            pltpu.async_copy(zt.at[idxrow.at[j]], gbuf.at[b], sems.at[b])

        def g_wait(b, j):
            pltpu.make_async_copy(zt.at[idxrow.at[j]], gbuf.at[b],
                                  sems.at[b]).wait()

        def s_issue(b, j):
            pltpu.async_copy(gbuf.at[b], acc.at[idxcol.at[j]], sems.at[b],
                             add=True)

        def s_wait(b, j):
            pltpu.make_async_copy(gbuf.at[b], acc.at[idxcol.at[j]],
                                  sems.at[b]).wait()

        pltpu.sync_copy(rowr.at[pl.ds(sid * _NCHUNK, _NCHUNK)], idxrow)
        pltpu.sync_copy(colr.at[pl.ds(sid * _NCHUNK, _NCHUNK)], idxcol)
        pltpu.sync_copy(zt.at[pl.ds(sid * _RPT, _RPT)],
                        acc.at[pl.ds(sid * _RPT, _RPT)])
        for b in range(_LOOK):
            g_issue(b, b)
        plsc.subcore_barrier()

        def chunk_static(j):
            u = j % _NB
            jn = j + _LOOK
            bn = jn % _NB
            if jn < _NCHUNK:
                if jn >= _NB:
                    s_wait(bn, jn - _NB)
                g_issue(bn, jn)
            g_wait(u, j)
            s_issue(u, j)

        ngrp = _NCHUNK // _NB

        for j in range(_NB):
            chunk_static(j)

        def grp(p, carry):
            j0 = p * _NB
            for u in range(_NB):
                j = j0 + u
                bn = (u + _LOOK) % _NB
                s_wait(bn, j + _LOOK - _NB)
                g_issue(bn, j + _LOOK)
                g_wait(u, j)
                s_issue(u, j)
            return carry

        lax.fori_loop(1, ngrp - 1, grp, 0)

        for j in range((ngrp - 1) * _NB, _NCHUNK):
            chunk_static(j)
        for b in range(_NB):
            last = max(j for j in range(_NCHUNK) if j % _NB == b)
            s_wait(b, last)

        plsc.subcore_barrier()
        pltpu.sync_copy(acc.at[pl.ds(sid * _RPT, _RPT)],
                        agg.at[pl.ds(sid * _RPT, _RPT)])

    @pl.when(cid == 0)
    def _():
        run(ztA, aggA)

    @pl.when(cid == 1)
    def _():
        run(ztB, aggB)


@functools.cache
def _sc_kernels():
    mesh = plsc.VectorSubcoreMesh(core_axis_name="c", subcore_axis_name="s")
    params = pltpu.CompilerParams(needs_layout_passes=False,
                                  use_tc_tiling_on_sc=False)
    deg = pl.kernel(
        _deg_body,
        out_type=jax.ShapeDtypeStruct((_NW, _N), jnp.float32),
        mesh=mesh,
        compiler_params=params,
        scratch_types=[
            pltpu.VMEM((_EPW + 8,), jnp.int32),
            pltpu.VMEM((_N,), jnp.float32),
        ],
    )
    agg = pl.kernel(
        _agg_body,
        out_type=[
            jax.ShapeDtypeStruct((_N, _H2), jnp.float32),
            jax.ShapeDtypeStruct((_N, _H2), jnp.float32),
        ],
        mesh=mesh,
        compiler_params=params,
        scratch_types=[
            pltpu.VMEM((_NCHUNK, _K), jnp.int32),
            pltpu.VMEM((_NCHUNK, _K), jnp.int32),
            pltpu.VMEM((_NB, _K, _H2), jnp.float32),
            pltpu.VMEM_SHARED((_N, _H2), jnp.float32),
            pltpu.SemaphoreType.DMA((_NB,)),
        ],
    )
    return deg, agg


def _t1_body(x_ref, w0_ref, parts_ref, ztA_ref, ztB_ref, dinv_ref):
    deg = jnp.sum(parts_ref[...], axis=0) + 1.0
    dinv = lax.rsqrt(deg)
    z = jnp.dot(x_ref[...], w0_ref[...], preferred_element_type=jnp.float32)
    zt = z * dinv[:, None]
    ztA_ref[...] = zt[:, :_H2]
    ztB_ref[...] = zt[:, _H2:]
    dinv_ref[...] = dinv[:, None]


def _t1(x, W0, parts):
    return pl.pallas_call(
        _t1_body,
        grid=(_G,),
        in_specs=[
            pl.BlockSpec((_B, _D), lambda i: (i, 0)),
            pl.BlockSpec((_D, _H), lambda i: (0, 0)),
            pl.BlockSpec((_NW, _B), lambda i: (0, i)),
        ],
        out_specs=[
            pl.BlockSpec((_B, _H2), lambda i: (i, 0)),
            pl.BlockSpec((_B, _H2), lambda i: (i, 0)),
            pl.BlockSpec((_B, 1), lambda i: (i, 0)),
        ],
        out_shape=[
            jax.ShapeDtypeStruct((_N, _H2), jnp.float32),
            jax.ShapeDtypeStruct((_N, _H2), jnp.float32),
            jax.ShapeDtypeStruct((_N, 1), jnp.float32),
        ],
    )(x, W0, parts)


def _t2_body(aggA_ref, aggB_ref, dinv_ref, b_ref, a_ref, w1_ref,
             ztA_ref, ztB_ref):
    agg = jnp.concatenate([aggA_ref[...], aggB_ref[...]], axis=1)
    dinv = dinv_ref[...]
    v = agg * dinv + b_ref[...]
    h = jnp.where(v >= 0, v, a_ref[...] * v)
    z = jnp.dot(h, w1_ref[...], preferred_element_type=jnp.float32)
    zt = z * dinv
    ztA_ref[...] = zt[:, :_H2]
    ztB_ref[...] = zt[:, _H2:]


def _t2(aggA, aggB, dinv, b0, a, W1):
    return pl.pallas_call(
        _t2_body,
        grid=(_G,),
        in_specs=[
            pl.BlockSpec((_B, _H2), lambda i: (i, 0)),
            pl.BlockSpec((_B, _H2), lambda i: (i, 0)),
            pl.BlockSpec((_B, 1), lambda i: (i, 0)),
            pl.BlockSpec((1, _H), lambda i: (0, 0)),
            pl.BlockSpec((1, _H), lambda i: (0, 0)),
            pl.BlockSpec((_H, _H), lambda i: (0, 0)),
        ],
        out_specs=[
            pl.BlockSpec((_B, _H2), lambda i: (i, 0)),
            pl.BlockSpec((_B, _H2), lambda i: (i, 0)),
        ],
        out_shape=[
            jax.ShapeDtypeStruct((_N, _H2), jnp.float32),
            jax.ShapeDtypeStruct((_N, _H2), jnp.float32),
        ],
    )(aggA, aggB, dinv, b0, a, W1)


def _t3_body(aggA_ref, aggB_ref, dinv_ref, b_ref, a_ref, out_ref):
    agg = jnp.concatenate([aggA_ref[...], aggB_ref[...]], axis=1)
    v = agg * dinv_ref[...] + b_ref[...]
    out_ref[...] = jnp.where(v >= 0, v, a_ref[...] * v)


def _t3(aggA, aggB, dinv, b1, a):
    return pl.pallas_call(
        _t3_body,
        grid=(_G,),
        in_specs=[
            pl.BlockSpec((_B, _H2), lambda i: (i, 0)),
            pl.BlockSpec((_B, _H2), lambda i: (i, 0)),
            pl.BlockSpec((_B, 1), lambda i: (i, 0)),
            pl.BlockSpec((1, _H), lambda i: (0, 0)),
            pl.BlockSpec((1, _H), lambda i: (0, 0)),
        ],
        out_specs=pl.BlockSpec((_B, _H), lambda i: (i, 0)),
        out_shape=jax.ShapeDtypeStruct((_N, _H), jnp.float32),
    )(aggA, aggB, dinv, b1, a)


def kernel(x, edge_index, W0, b0, W1, b1, prelu_a):
    deg_call, agg_call = _sc_kernels()
    row = edge_index[0]
    col = edge_index[1]
    rowr = row.reshape(_NS * _NCHUNK, _K)
    colr = col.reshape(_NS * _NCHUNK, _K)

    parts = deg_call(col)
    ztA0, ztB0, dinv = _t1(x, W0, parts)
    aggA0, aggB0 = agg_call(ztA0, ztB0, rowr, colr)
    ztA1, ztB1 = _t2(aggA0, aggB0, dinv, b0[None], prelu_a[None], W1)
    aggA1, aggB1 = agg_call(ztA1, ztB1, rowr, colr)
    return _t3(aggA1, aggB1, dinv, b1[None], prelu_a[None])


# speedup vs baseline: 1.0122x; 1.0122x over previous
import functools

import jax
import jax.numpy as jnp
from jax import lax
from jax.experimental import pallas as pl
from jax.experimental.pallas import tpu as pltpu
from jax.experimental.pallas import tpu_sc as plsc

_N = 10000
_E = 160000
_D = 256
_H = 256
_H2 = _H // 2

_NC = 2
_NS = 16
_NW = _NC * _NS
_K = 80
_EPT = _E // _NS
_NCHUNK = _EPT // _K
_RPT = _N // _NS
_EPW = _E // _NW

_B = 1024
_G = (_N + _B - 1) // _B


def _deg_body(col_hbm, parts_hbm, colbuf, acc):
    cid = lax.axis_index("c")
    sid = lax.axis_index("s")
    wid = cid * _NS + sid

    zeros16 = jnp.zeros((16,), jnp.float32)

    def zstep(j, carry):
        acc[pl.ds(j * 16, 16)] = zeros16
        return carry

    lax.fori_loop(0, _N // 16, zstep, 0)

    colbuf[pl.ds(_EPW - 8, 16)] = jnp.zeros((16,), jnp.int32)
    pltpu.sync_copy(col_hbm.at[pl.ds(wid * _EPW, _EPW)], colbuf.at[pl.ds(0, _EPW)])

    ones16 = jnp.ones((16,), jnp.float32)
    nfull = _EPW // 16

    def estep(i, carry):
        idx = colbuf[pl.ds(i * 16, 16)]
        plsc.addupdate_scatter(acc, [idx], ones16)
        return carry

    lax.fori_loop(0, nfull, estep, 0)

    rem = _EPW - nfull * 16
    idx = colbuf[pl.ds(nfull * 16, 16)]
    mask = lax.iota(jnp.int32, 16) < rem
    plsc.addupdate_scatter(acc, [idx], ones16, mask=mask)

    pltpu.sync_copy(acc, parts_hbm.at[wid])


_NB = 3
_LOOK = 1


def _agg_body(ztA, ztB, rowr, colr, aggA, aggB, idxrow, idxcol, gbuf, acc,
              sems):
    cid = lax.axis_index("c")
    sid = lax.axis_index("s")

    def run(zt, agg):
        def g_issue(b, j):
            pltpu.async_copy(zt.at[idxrow.at[j]], gbuf.at[b], sems.at[b])

        def g_wait(b, j):
            pltpu.make_async_copy(zt.at[idxrow.at[j]], gbuf.at[b],
                                  sems.at[b]).wait()

        def s_issue(b, j):
            pltpu.async_copy(gbuf.at[b], acc.at[idxcol.at[j]], sems.at[b],
                             add=True)

        def s_wait(b, j):
            pltpu.make_async_copy(gbuf.at[b], acc.at[idxcol.at[j]],
                                  sems.at[b]).wait()

        cp_r = pltpu.async_copy(rowr.at[pl.ds(sid * _NCHUNK, _NCHUNK)],
                                idxrow, sems.at[0])
        cp_c = pltpu.async_copy(colr.at[pl.ds(sid * _NCHUNK, _NCHUNK)],
                                idxcol, sems.at[1])
        cp_i = pltpu.async_copy(zt.at[pl.ds(sid * _RPT, _RPT)],
                                acc.at[pl.ds(sid * _RPT, _RPT)], sems.at[2])
        cp_r.wait()
        for b in range(_LOOK):
            g_issue(b, b)
        cp_c.wait()
        cp_i.wait()
        plsc.subcore_barrier()

        def chunk_static(j):
            u = j % _NB
            jn = j + _LOOK
            bn = jn % _NB
            if jn < _NCHUNK:
                if jn >= _NB:
                    s_wait(bn, jn - _NB)
                g_issue(bn, jn)
            g_wait(u, j)
            s_issue(u, j)

        ngrp = _NCHUNK // _NB

        for j in range(_NB):
            chunk_static(j)

        def grp(p, carry):
            j0 = p * _NB
            for u in range(_NB):
                j = j0 + u
                bn = (u + _LOOK) % _NB
                s_wait(bn, j + _LOOK - _NB)
                g_issue(bn, j + _LOOK)
                g_wait(u, j)
                s_issue(u, j)
            return carry

        lax.fori_loop(1, ngrp - 1, grp, 0)

        for j in range((ngrp - 1) * _NB, _NCHUNK):
            chunk_static(j)
        for b in range(_NB):
            last = max(j for j in range(_NCHUNK) if j % _NB == b)
            s_wait(b, last)

        plsc.subcore_barrier()
        pltpu.sync_copy(acc.at[pl.ds(sid * _RPT, _RPT)],
                        agg.at[pl.ds(sid * _RPT, _RPT)])

    @pl.when(cid == 0)
    def _():
        run(ztA, aggA)

    @pl.when(cid == 1)
    def _():
        run(ztB, aggB)


@functools.cache
def _sc_kernels():
    mesh = plsc.VectorSubcoreMesh(core_axis_name="c", subcore_axis_name="s")
    params = pltpu.CompilerParams(needs_layout_passes=False,
                                  use_tc_tiling_on_sc=False)
    deg = pl.kernel(
        _deg_body,
        out_type=jax.ShapeDtypeStruct((_NW, _N), jnp.float32),
        mesh=mesh,
        compiler_params=params,
        scratch_types=[
            pltpu.VMEM((_EPW + 8,), jnp.int32),
            pltpu.VMEM((_N,), jnp.float32),
        ],
    )
    agg = pl.kernel(
        _agg_body,
        out_type=[
            jax.ShapeDtypeStruct((_N, _H2), jnp.float32),
            jax.ShapeDtypeStruct((_N, _H2), jnp.float32),
        ],
        mesh=mesh,
        compiler_params=params,
        scratch_types=[
            pltpu.VMEM((_NCHUNK, _K), jnp.int32),
            pltpu.VMEM((_NCHUNK, _K), jnp.int32),
            pltpu.VMEM((_NB, _K, _H2), jnp.float32),
            pltpu.VMEM_SHARED((_N, _H2), jnp.float32),
            pltpu.SemaphoreType.DMA((_NB,)),
        ],
    )
    return deg, agg


def _t1_body(x_ref, w0_ref, parts_ref, ztA_ref, ztB_ref, dinv_ref):
    deg = jnp.sum(parts_ref[...], axis=0) + 1.0
    dinv = lax.rsqrt(deg)
    z = jnp.dot(x_ref[...], w0_ref[...], preferred_element_type=jnp.float32)
    zt = z * dinv[:, None]
    ztA_ref[...] = zt[:, :_H2]
    ztB_ref[...] = zt[:, _H2:]
    dinv_ref[...] = dinv[:, None]


def _t1(x, W0, parts):
    return pl.pallas_call(
        _t1_body,
        grid=(_G,),
        in_specs=[
            pl.BlockSpec((_B, _D), lambda i: (i, 0)),
            pl.BlockSpec((_D, _H), lambda i: (0, 0)),
            pl.BlockSpec((_NW, _B), lambda i: (0, i)),
        ],
        out_specs=[
            pl.BlockSpec((_B, _H2), lambda i: (i, 0)),
            pl.BlockSpec((_B, _H2), lambda i: (i, 0)),
            pl.BlockSpec((_B, 1), lambda i: (i, 0)),
        ],
        out_shape=[
            jax.ShapeDtypeStruct((_N, _H2), jnp.float32),
            jax.ShapeDtypeStruct((_N, _H2), jnp.float32),
            jax.ShapeDtypeStruct((_N, 1), jnp.float32),
        ],
    )(x, W0, parts)


def _t2_body(aggA_ref, aggB_ref, dinv_ref, b_ref, a_ref, w1_ref,
             ztA_ref, ztB_ref):
    agg = jnp.concatenate([aggA_ref[...], aggB_ref[...]], axis=1)
    dinv = dinv_ref[...]
    v = agg * dinv + b_ref[...]
    h = jnp.where(v >= 0, v, a_ref[...] * v)
    z = jnp.dot(h, w1_ref[...], preferred_element_type=jnp.float32)
    zt = z * dinv
    ztA_ref[...] = zt[:, :_H2]
    ztB_ref[...] = zt[:, _H2:]


def _t2(aggA, aggB, dinv, b0, a, W1):
    return pl.pallas_call(
        _t2_body,
        grid=(_G,),
        in_specs=[
            pl.BlockSpec((_B, _H2), lambda i: (i, 0)),
            pl.BlockSpec((_B, _H2), lambda i: (i, 0)),
            pl.BlockSpec((_B, 1), lambda i: (i, 0)),
            pl.BlockSpec((1, _H), lambda i: (0, 0)),
            pl.BlockSpec((1, _H), lambda i: (0, 0)),
            pl.BlockSpec((_H, _H), lambda i: (0, 0)),
        ],
        out_specs=[
            pl.BlockSpec((_B, _H2), lambda i: (i, 0)),
            pl.BlockSpec((_B, _H2), lambda i: (i, 0)),
        ],
        out_shape=[
            jax.ShapeDtypeStruct((_N, _H2), jnp.float32),
            jax.ShapeDtypeStruct((_N, _H2), jnp.float32),
        ],
    )(aggA, aggB, dinv, b0, a, W1)


def _t3_body(aggA_ref, aggB_ref, dinv_ref, b_ref, a_ref, out_ref):
    agg = jnp.concatenate([aggA_ref[...], aggB_ref[...]], axis=1)
    v = agg * dinv_ref[...] + b_ref[...]
    out_ref[...] = jnp.where(v >= 0, v, a_ref[...] * v)


def _t3(aggA, aggB, dinv, b1, a):
    return pl.pallas_call(
        _t3_body,
        grid=(_G,),
        in_specs=[
            pl.BlockSpec((_B, _H2), lambda i: (i, 0)),
            pl.BlockSpec((_B, _H2), lambda i: (i, 0)),
            pl.BlockSpec((_B, 1), lambda i: (i, 0)),
            pl.BlockSpec((1, _H), lambda i: (0, 0)),
            pl.BlockSpec((1, _H), lambda i: (0, 0)),
        ],
        out_specs=pl.BlockSpec((_B, _H), lambda i: (i, 0)),
        out_shape=jax.ShapeDtypeStruct((_N, _H), jnp.float32),
    )(aggA, aggB, dinv, b1, a)


def kernel(x, edge_index, W0, b0, W1, b1, prelu_a):
    deg_call, agg_call = _sc_kernels()
    row = edge_index[0]
    col = edge_index[1]
    rowr = row.reshape(_NS * _NCHUNK, _K)
    colr = col.reshape(_NS * _NCHUNK, _K)

    parts = deg_call(col)
    ztA0, ztB0, dinv = _t1(x, W0, parts)
    aggA0, aggB0 = agg_call(ztA0, ztB0, rowr, colr)
    ztA1, ztB1 = _t2(aggA0, aggB0, dinv, b0[None], prelu_a[None], W1)
    aggA1, aggB1 = agg_call(ztA1, ztB1, rowr, colr)
    return _t3(aggA1, aggB1, dinv, b1[None], prelu_a[None])
